# baseline (speedup 1.0000x reference)
import functools

import jax
import jax.numpy as jnp
from jax import lax
from jax.experimental import pallas as pl
from jax.experimental.pallas import tpu as pltpu
from jax.experimental.pallas import tpu_sc as plsc

N = 10000
E = 320000
IN = 16
OUT = 16
D_EDGE = 16
HID = 64

NC = 2
NS = 16
NW = NC * NS
LANE = 128
RPW = 80
E_PAD = NW * RPW * LANE
PAD = E_PAD - E

N_ACC = 10016
STRIPE = N_ACC // NS

G_CH = 4
GA = 120
GB = 40
GIDX = NS * GA + NS * GB + (GA - GB)
S_CH = 8

BE = 2048

_f32 = jnp.float32
_bf16 = jnp.bfloat16


def _gather_body(x_hbm, srcidx_hbm, xj_hbm, idx_v, gbuf0, gbuf1, gsem,
                 osem0, osem1):
    c = lax.axis_index("c")
    s = lax.axis_index("s")
    base_row = jnp.where(c == 0, s * GA, NS * GA + s * GB)
    n_outer = jnp.where(c == 0, GA // (2 * G_CH), GB // (2 * G_CH))
    pltpu.sync_copy(srcidx_hbm.at[pl.ds(base_row, GA)], idx_v)
    gbufs = (gbuf0, gbuf1)
    osems = (osem0, osem1)

    def outer(k2, carry):
        for b in range(2):
            kk = k2 * 2 + b
            gb = gbufs[b]
            os_ = osems[b]

            @pl.when(kk >= 2)
            def _drain():
                pltpu.make_async_copy(
                    gb, xj_hbm.at[pl.ds(0, G_CH * LANE)], os_).wait()

            descs = []
            for r in range(G_CH):
                descs.append(
                    pltpu.async_copy(
                        x_hbm.at[idx_v.at[kk * G_CH + r]],
                        gb.at[pl.ds(r * LANE, LANE)],
                        gsem,
                    )
                )
            for d in descs:
                d.wait()
            pltpu.async_copy(
                gb,
                xj_hbm.at[pl.ds((base_row + kk * G_CH) * LANE, G_CH * LANE)],
                os_,
            )
        return carry

    lax.fori_loop(0, n_outer, outer, 0)
    for b in range(2):
        pltpu.make_async_copy(
            gbufs[b], xj_hbm.at[pl.ds(0, G_CH * LANE)], osems[b]).wait()


_gather = functools.partial(
    pl.kernel,
    out_type=jax.ShapeDtypeStruct((E_PAD, IN), _f32),
    mesh=plsc.VectorSubcoreMesh(core_axis_name="c", subcore_axis_name="s"),
    scratch_types=[
        pltpu.VMEM((GA, LANE), jnp.int32),
        pltpu.VMEM((G_CH * LANE, IN), _f32),
        pltpu.VMEM((G_CH * LANE, IN), _f32),
        pltpu.SemaphoreType.DMA,
        pltpu.SemaphoreType.DMA,
        pltpu.SemaphoreType.DMA,
    ],
    compiler_params=pltpu.CompilerParams(use_tc_tiling_on_sc=False),
)(_gather_body)


def _msgs_body(ea, xj, w1, b1, w2, b2, rmat, smat, out):
    h = jnp.maximum(
        jnp.dot(ea[...], w1[...], preferred_element_type=_f32) + b1[...], 0.0
    )
    wflat = jnp.dot(h, w2[...], preferred_element_type=_f32) + b2[...]
    xt = jnp.dot(xj[...], rmat[...], preferred_element_type=_f32)
    out[...] = jnp.dot(xt * wflat, smat[...], preferred_element_type=_f32)


def _msgs(ea_p, xj, W1, b1, W2, b2, rmat, smat):
    grid = (E_PAD // BE,)
    full = lambda shape: pl.BlockSpec(shape, lambda i: (0, 0))
    return pl.pallas_call(
        _msgs_body,
        grid=grid,
        in_specs=[
            pl.BlockSpec((BE, D_EDGE), lambda i: (i, 0)),
            pl.BlockSpec((BE, IN), lambda i: (i, 0)),
            full((D_EDGE, HID)),
            full((1, HID)),
            full((HID, IN * OUT)),
            full((1, IN * OUT)),
            full((IN, IN * OUT)),
            full((IN * OUT, OUT)),
        ],
        out_specs=pl.BlockSpec((BE, OUT), lambda i: (i, 0)),
        out_shape=jax.ShapeDtypeStruct((E_PAD, OUT), _f32),
        compiler_params=pltpu.CompilerParams(
            dimension_semantics=("arbitrary",)
        ),
    )(ea_p, xj, W1, b1, W2, b2, rmat, smat)


def _counts_body(dstidx_hbm, zeros_hbm, ones_hbm, cnts_hbm,
                 idx_v, onesb, cacc, csem):
    c = lax.axis_index("c")
    s = lax.axis_index("s")
    w = s * NC + c
    pltpu.sync_copy(zeros_hbm.at[pl.ds(s * STRIPE, STRIPE)],
                    cacc.at[pl.ds(s * STRIPE, STRIPE)])
    pltpu.sync_copy(ones_hbm, onesb)
    pltpu.sync_copy(dstidx_hbm.at[w], idx_v)
    plsc.subcore_barrier()

    def chunk(k, carry):
        for r in range(S_CH):
            pltpu.async_copy(
                onesb, cacc.at[idx_v.at[k * S_CH + r]], csem, add=True)
        for r in range(S_CH):
            pltpu.make_async_copy(ones_hbm, onesb, csem).wait()
        return carry

    lax.fori_loop(0, RPW // S_CH, chunk, 0)
    plsc.subcore_barrier()
    pltpu.sync_copy(cacc.at[pl.ds(s * STRIPE, STRIPE)],
                    cnts_hbm.at[c, pl.ds(s * STRIPE, STRIPE)])


_counts = functools.partial(
    pl.kernel,
    out_type=jax.ShapeDtypeStruct((NC, N_ACC, OUT), _f32),
    mesh=plsc.VectorSubcoreMesh(core_axis_name="c", subcore_axis_name="s"),
    scratch_types=[
        pltpu.VMEM((RPW, LANE), jnp.int32),
        pltpu.VMEM((LANE, OUT), _f32),
        pltpu.VMEM_SHARED((N_ACC, OUT), _f32),
        pltpu.SemaphoreType.DMA,
    ],
    compiler_params=pltpu.CompilerParams(use_tc_tiling_on_sc=False),
)(_counts_body)


def _scatter_body(msgs_hbm, dstidx_hbm, zeros_hbm, sums_hbm,
                  idx_v, mbuf0, mbuf1, acc, ssem0, ssem1):
    c = lax.axis_index("c")
    s = lax.axis_index("s")
    w = s * NC + c
    pltpu.sync_copy(zeros_hbm.at[pl.ds(s * STRIPE, STRIPE)],
                    acc.at[pl.ds(s * STRIPE, STRIPE)])
    pltpu.sync_copy(dstidx_hbm.at[w], idx_v)
    plsc.subcore_barrier()
    mbufs = (mbuf0, mbuf1)
    ssems = (ssem0, ssem1)

    def outer(k2, carry):
        for b in range(2):
            kk = k2 * 2 + b
            mb = mbufs[b]
            ss = ssems[b]

            @pl.when(kk >= 2)
            def _drain():
                pltpu.make_async_copy(
                    msgs_hbm.at[pl.ds(0, S_CH * LANE)], mb, ss).wait()

            pltpu.sync_copy(
                msgs_hbm.at[pl.ds((w * RPW + kk * S_CH) * LANE,
                                  S_CH * LANE)], mb)
            for r in range(S_CH):
                pltpu.async_copy(
                    mb.at[pl.ds(r * LANE, LANE)],
                    acc.at[idx_v.at[kk * S_CH + r]],
                    ss,
                    add=True,
                )
        return carry

    lax.fori_loop(0, RPW // S_CH // 2, outer, 0)
    for b in range(2):
        pltpu.make_async_copy(
            msgs_hbm.at[pl.ds(0, S_CH * LANE)], mbufs[b], ssems[b]).wait()
    plsc.subcore_barrier()
    pltpu.sync_copy(acc.at[pl.ds(s * STRIPE, STRIPE)],
                    sums_hbm.at[c, pl.ds(s * STRIPE, STRIPE)])


_scatter = functools.partial(
    pl.kernel,
    out_type=jax.ShapeDtypeStruct((NC, N_ACC, OUT), _f32),
    mesh=plsc.VectorSubcoreMesh(core_axis_name="c", subcore_axis_name="s"),
    scratch_types=[
        pltpu.VMEM((RPW, LANE), jnp.int32),
        pltpu.VMEM((S_CH * LANE, OUT), _f32),
        pltpu.VMEM((S_CH * LANE, OUT), _f32),
        pltpu.VMEM_SHARED((N_ACC, OUT), _f32),
        pltpu.SemaphoreType.DMA,
        pltpu.SemaphoreType.DMA,
    ],
    compiler_params=pltpu.CompilerParams(use_tc_tiling_on_sc=False),
)(_scatter_body)


def _final_body(s0, s1, c0, c1, x_ref, root_ref, bias_ref, gamma_ref,
                beta_ref, out_ref):
    summ = s0[...] + s1[...]
    cnt = c0[...] + c1[...]
    summ = summ[0:N]
    cnt = cnt[0:N]
    aggr = summ / jnp.maximum(cnt, 1.0)
    xv = x_ref[...]
    h = aggr + jnp.dot(xv, root_ref[...], preferred_element_type=_f32) \
        + bias_ref[...]
    mu = jnp.mean(h, axis=0, keepdims=True)
    var = jnp.mean((h - mu) ** 2, axis=0, keepdims=True)
    hn = (h - mu) / jnp.sqrt(var + 1e-5) * gamma_ref[...] + beta_ref[...]
    out_ref[...] = xv + jnp.maximum(hn, 0.0)


def _final(s0, s1, c0, c1, x, root, bias, gamma, beta):
    return pl.pallas_call(
        _final_body,
        out_shape=jax.ShapeDtypeStruct((N, OUT), _f32),
    )(s0, s1, c0, c1, x, root, bias, gamma, beta)


def kernel(x, edge_index, edge_attr, W1, b1, W2, b2, root, bias, gamma, beta):
    src = edge_index[0].astype(jnp.int32)
    dst = edge_index[1].astype(jnp.int32)
    src_p = jnp.concatenate(
        [src, jnp.zeros((GIDX * LANE - E,), jnp.int32)]).reshape(GIDX, LANE)
    dst_p = jnp.concatenate(
        [dst, jnp.full((PAD,), N, jnp.int32)]).reshape(NW, RPW, LANE)
    ea_p = jnp.concatenate(
        [edge_attr, jnp.zeros((PAD, D_EDGE), _f32)], axis=0)

    cols = jnp.arange(IN * OUT)
    rmat = (cols[None, :] // OUT == jnp.arange(IN)[:, None]).astype(_f32)
    smat = (cols[:, None] % OUT == jnp.arange(OUT)[None, :]).astype(_f32)

    zeros_c = jnp.zeros((N_ACC, OUT), _f32)
    ones_c = jnp.ones((LANE, OUT), _f32)

    cnts = _counts(dst_p, zeros_c, ones_c)
    xj = _gather(x, src_p)
    msgs = _msgs(ea_p, xj, W1, b1.reshape(1, HID),
                 W2, b2.reshape(1, IN * OUT), rmat, smat)
    sums = _scatter(msgs, dst_p, zeros_c)

    return _final(sums[0], sums[1], cnts[0], cnts[1], x, root,
                  bias.reshape(1, OUT), gamma.reshape(1, OUT),
                  beta.reshape(1, OUT))


# speedup vs baseline: 1.5815x; 1.5815x over previous
import functools

import jax
import jax.numpy as jnp
from jax import lax
from jax.experimental import pallas as pl
from jax.experimental.pallas import tpu as pltpu
from jax.experimental.pallas import tpu_sc as plsc

N = 10000
E = 320000
IN = 16
OUT = 16
D_EDGE = 16
HID = 64

NC = 2
NS = 16
NW = NC * NS
LANE = 128
RPW = 80
E_PAD = NW * RPW * LANE
EW = E_PAD // 8
PAD = E_PAD - E

G_CH = 4
GA = 120
GB = 40
GIDX = NS * GA + NS * GB + (GA - GB)
S_CH = 8

N_ACC = 10240
NAW = N_ACC * 16 // 128
STRIPE = N_ACC // NS
SW = STRIPE * 16 // 128

BE = 2048
BEW = BE // 8
NB = (E + BE - 1) // BE

_f32 = jnp.float32


def _gather_body(x_hbm, srcidx_hbm, xj_hbm, idx_v, gbuf0, gbuf1, gsem,
                 osem0, osem1):
    c = lax.axis_index("c")
    s = lax.axis_index("s")
    base_row = jnp.where(c == 0, s * GA, NS * GA + s * GB)
    n_outer = jnp.where(c == 0, GA // (2 * G_CH), GB // (2 * G_CH))
    pltpu.sync_copy(srcidx_hbm.at[pl.ds(base_row, GA)], idx_v)
    gbufs = (gbuf0, gbuf1)
    osems = (osem0, osem1)

    def outer(k2, carry):
        for b in range(2):
            kk = k2 * 2 + b
            gb = gbufs[b]
            os_ = osems[b]

            @pl.when(kk >= 2)
            def _drain():
                pltpu.make_async_copy(
                    gb, xj_hbm.at[pl.ds(0, G_CH * LANE), pl.ds(0, IN)],
                    os_).wait()

            descs = []
            for r in range(G_CH):
                descs.append(
                    pltpu.async_copy(
                        x_hbm.at[idx_v.at[kk * G_CH + r]],
                        gb.at[pl.ds(r * LANE, LANE)],
                        gsem,
                    )
                )
            for d in descs:
                d.wait()
            pltpu.async_copy(
                gb,
                xj_hbm.at[pl.ds((base_row + kk * G_CH) * LANE, G_CH * LANE),
                          pl.ds(0, IN)],
                os_,
            )
        return carry

    lax.fori_loop(0, n_outer, outer, 0)
    for b in range(2):
        pltpu.make_async_copy(
            gbufs[b], xj_hbm.at[pl.ds(0, G_CH * LANE), pl.ds(0, IN)],
            osems[b]).wait()


_gather = functools.partial(
    pl.kernel,
    out_type=jax.ShapeDtypeStruct((E_PAD, 128), _f32),
    mesh=plsc.VectorSubcoreMesh(core_axis_name="c", subcore_axis_name="s"),
    scratch_types=[
        pltpu.VMEM((GA, LANE), jnp.int32),
        pltpu.VMEM((G_CH * LANE, IN), _f32),
        pltpu.VMEM((G_CH * LANE, IN), _f32),
        pltpu.SemaphoreType.DMA,
        pltpu.SemaphoreType.DMA,
        pltpu.SemaphoreType.DMA,
    ],
    compiler_params=pltpu.CompilerParams(use_tc_tiling_on_sc=False),
)(_gather_body)


def _msgs_body(ea, xj, w1, b1, w2, b2, rmat, smat, out):
    h = jnp.maximum(
        jnp.dot(ea[...], w1[...], preferred_element_type=_f32) + b1[...], 0.0
    )
    wflat = jnp.dot(h, w2[...], preferred_element_type=_f32) + b2[...]
    xt = jnp.dot(xj[:, 0:IN], rmat[...], preferred_element_type=_f32)
    out[:, 0:OUT] = jnp.dot(xt * wflat, smat[...],
                            preferred_element_type=_f32)


def _msgs(ea, xj, W1, b1, W2, b2, rmat, smat):
    full = lambda shape: pl.BlockSpec(shape, lambda i: (0, 0))
    return pl.pallas_call(
        _msgs_body,
        grid=(NB,),
        in_specs=[
            pl.BlockSpec((BE, D_EDGE), lambda i: (i, 0)),
            pl.BlockSpec((BE, 128), lambda i: (i, 0)),
            full((D_EDGE, HID)),
            full((1, HID)),
            full((HID, IN * OUT)),
            full((1, IN * OUT)),
            full((IN, IN * OUT)),
            full((IN * OUT, OUT)),
        ],
        out_specs=pl.BlockSpec((BE, 128), lambda i: (i, 0)),
        out_shape=jax.ShapeDtypeStruct((E_PAD, 128), _f32),
        compiler_params=pltpu.CompilerParams(
            dimension_semantics=("arbitrary",)
        ),
    )(ea, xj, W1, b1, W2, b2, rmat, smat)


def _counts_body(dstidx_hbm, cnts_hbm, idx_v, onesb, zbuf, cacc, csem):
    c = lax.axis_index("c")
    s = lax.axis_index("s")
    w = s * NC + c

    def fill(i, carry):
        zbuf[i] = jnp.zeros((OUT,), _f32)
        return carry

    lax.fori_loop(0, STRIPE, fill, 0)

    def fill1(i, carry):
        onesb[i] = jnp.ones((OUT,), _f32)
        return carry

    lax.fori_loop(0, LANE, fill1, 0)
    pltpu.sync_copy(zbuf, cacc.at[pl.ds(s * STRIPE, STRIPE)])
    pltpu.sync_copy(dstidx_hbm.at[w], idx_v)
    plsc.subcore_barrier()

    def chunk(k, carry):
        for r in range(S_CH):
            pltpu.async_copy(
                onesb, cacc.at[idx_v.at[k * S_CH + r]], csem, add=True)
        for r in range(S_CH):
            pltpu.make_async_copy(
                cnts_hbm.at[0, pl.ds(0, LANE)], onesb, csem).wait()
        return carry

    lax.fori_loop(0, RPW // S_CH, chunk, 0)
    plsc.subcore_barrier()
    pltpu.sync_copy(cacc.at[pl.ds(s * STRIPE, STRIPE)],
                    cnts_hbm.at[c, pl.ds(s * STRIPE, STRIPE)])


_counts = functools.partial(
    pl.kernel,
    out_type=jax.ShapeDtypeStruct((NC, N_ACC, OUT), _f32),
    mesh=plsc.VectorSubcoreMesh(core_axis_name="c", subcore_axis_name="s"),
    scratch_types=[
        pltpu.VMEM((RPW, LANE), jnp.int32),
        pltpu.VMEM((LANE, OUT), _f32),
        pltpu.VMEM((STRIPE, OUT), _f32),
        pltpu.VMEM_SHARED((N_ACC, OUT), _f32),
        pltpu.SemaphoreType.DMA,
    ],
    compiler_params=pltpu.CompilerParams(use_tc_tiling_on_sc=False),
)(_counts_body)


def _scatter_body(msgs_hbm, dstidx_hbm, sums_hbm,
                  idx_v, mbuf0, mbuf1, zbuf, acc, ssem0, ssem1):
    c = lax.axis_index("c")
    s = lax.axis_index("s")
    w = s * NC + c

    def fill(i, carry):
        zbuf[i] = jnp.zeros((OUT,), _f32)
        return carry

    lax.fori_loop(0, STRIPE, fill, 0)
    pltpu.sync_copy(zbuf, acc.at[pl.ds(s * STRIPE, STRIPE)])
    pltpu.sync_copy(dstidx_hbm.at[w], idx_v)
    plsc.subcore_barrier()
    mbufs = (mbuf0, mbuf1)
    ssems = (ssem0, ssem1)

    def outer(k2, carry):
        for b in range(2):
            kk = k2 * 2 + b
            mb = mbufs[b]
            ss = ssems[b]

            @pl.when(kk >= 2)
            def _drain():
                pltpu.make_async_copy(
                    msgs_hbm.at[pl.ds(0, S_CH * LANE), pl.ds(0, OUT)],
                    mb, ss).wait()

            pltpu.sync_copy(
                msgs_hbm.at[pl.ds((w * RPW + kk * S_CH) * LANE,
                                  S_CH * LANE), pl.ds(0, OUT)], mb)
            for r in range(S_CH):
                pltpu.async_copy(
                    mb.at[pl.ds(r * LANE, LANE)],
                    acc.at[idx_v.at[kk * S_CH + r]],
                    ss,
                    add=True,
                )
        return carry

    lax.fori_loop(0, RPW // S_CH // 2, outer, 0)
    for b in range(2):
        pltpu.make_async_copy(
            msgs_hbm.at[pl.ds(0, S_CH * LANE), pl.ds(0, OUT)],
            mbufs[b], ssems[b]).wait()
    plsc.subcore_barrier()
    pltpu.sync_copy(acc.at[pl.ds(s * STRIPE, STRIPE)],
                    sums_hbm.at[c, pl.ds(s * STRIPE, STRIPE)])


_scatter = functools.partial(
    pl.kernel,
    out_type=jax.ShapeDtypeStruct((NC, N_ACC, OUT), _f32),
    mesh=plsc.VectorSubcoreMesh(core_axis_name="c", subcore_axis_name="s"),
    scratch_types=[
        pltpu.VMEM((RPW, LANE), jnp.int32),
        pltpu.VMEM((S_CH * LANE, OUT), _f32),
        pltpu.VMEM((S_CH * LANE, OUT), _f32),
        pltpu.VMEM((STRIPE, OUT), _f32),
        pltpu.VMEM_SHARED((N_ACC, OUT), _f32),
        pltpu.SemaphoreType.DMA,
        pltpu.SemaphoreType.DMA,
    ],
    compiler_params=pltpu.CompilerParams(use_tc_tiling_on_sc=False),
)(_scatter_body)


def _final_body(sums_ref, cnts_ref, x_ref, root_ref, bias_ref, gamma_ref,
                beta_ref, out_ref):
    summ = (sums_ref[0] + sums_ref[1])[0:N]
    cnt = (cnts_ref[0] + cnts_ref[1])[0:N]
    aggr = summ / jnp.maximum(cnt, 1.0)
    xv = x_ref[...]
    h = aggr + jnp.dot(xv, root_ref[...], preferred_element_type=_f32) \
        + bias_ref[...]
    mu = jnp.mean(h, axis=0, keepdims=True)
    var = jnp.mean((h - mu) ** 2, axis=0, keepdims=True)
    hn = (h - mu) / jnp.sqrt(var + 1e-5) * gamma_ref[...] + beta_ref[...]
    out_ref[...] = xv + jnp.maximum(hn, 0.0)


def _final(sums, cnts, x, root, bias, gamma, beta):
    return pl.pallas_call(
        _final_body,
        out_shape=jax.ShapeDtypeStruct((N, OUT), _f32),
    )(sums, cnts, x, root, bias, gamma, beta)


def kernel(x, edge_index, edge_attr, W1, b1, W2, b2, root, bias, gamma, beta):
    src = edge_index[0].astype(jnp.int32)
    dst = edge_index[1].astype(jnp.int32)
    src_p = jnp.concatenate(
        [src, jnp.zeros((GIDX * LANE - E,), jnp.int32)]).reshape(GIDX, LANE)
    dst_p = jnp.concatenate(
        [dst, jnp.full((PAD,), N, jnp.int32)]).reshape(NW, RPW, LANE)

    cols = jnp.arange(IN * OUT)
    rmat = (cols[None, :] // OUT == jnp.arange(IN)[:, None]).astype(_f32)
    smat = (cols[:, None] % OUT == jnp.arange(OUT)[None, :]).astype(_f32)

    cnts = _counts(dst_p)
    xj = _gather(x, src_p)
    msgs = _msgs(edge_attr, xj, W1, b1.reshape(1, HID), W2,
                 b2.reshape(1, IN * OUT), rmat, smat)
    sums = _scatter(msgs, dst_p)

    return _final(sums, cnts, x, root,
                  bias.reshape(1, OUT), gamma.reshape(1, OUT),
                  beta.reshape(1, OUT))


# speedup vs baseline: 1.7930x; 1.1338x over previous
import functools

import jax
import jax.numpy as jnp
from jax import lax
from jax.experimental import pallas as pl
from jax.experimental.pallas import tpu as pltpu
from jax.experimental.pallas import tpu_sc as plsc

N = 10000
E = 320000
IN = 16
OUT = 16
D_EDGE = 16
HID = 64

NC = 2
NS = 16
NW = NC * NS
LANE = 128
RPW = 80
E_PAD = NW * RPW * LANE
EW = E_PAD // 8
PAD = E_PAD - E

G_CH = 4
GA = 112
GB = 48
GIDX = NS * GA + NS * GB + (GA - GB)
S_CH = 8

N_ACC = 10240
NAW = N_ACC * 16 // 128
STRIPE = N_ACC // NS
SW = STRIPE * 16 // 128

BE = 4096
BEW = BE // 8
NB = (E + BE - 1) // BE

_f32 = jnp.float32


def _gather_body(x_hbm, srcidx_hbm, xj_hbm, idx_v, gbuf0, gbuf1, gsem,
                 osem0, osem1):
    c = lax.axis_index("c")
    s = lax.axis_index("s")
    base_row = jnp.where(c == 0, s * GA, NS * GA + s * GB)
    n_outer = jnp.where(c == 0, GA // (2 * G_CH), GB // (2 * G_CH))
    pltpu.sync_copy(srcidx_hbm.at[pl.ds(base_row, GA)], idx_v)
    gbufs = (gbuf0, gbuf1)
    osems = (osem0, osem1)

    def outer(k2, carry):
        for b in range(2):
            kk = k2 * 2 + b
            gb = gbufs[b]
            os_ = osems[b]

            @pl.when(kk >= 2)
            def _drain():
                pltpu.make_async_copy(
                    gb, xj_hbm.at[pl.ds(0, G_CH * LANE), pl.ds(0, IN)],
                    os_).wait()

            descs = []
            for r in range(G_CH):
                descs.append(
                    pltpu.async_copy(
                        x_hbm.at[idx_v.at[kk * G_CH + r]],
                        gb.at[pl.ds(r * LANE, LANE)],
                        gsem,
                    )
                )
            for d in descs:
                d.wait()
            pltpu.async_copy(
                gb,
                xj_hbm.at[pl.ds((base_row + kk * G_CH) * LANE, G_CH * LANE),
                          pl.ds(0, IN)],
                os_,
            )
        return carry

    lax.fori_loop(0, n_outer, outer, 0)
    for b in range(2):
        pltpu.make_async_copy(
            gbufs[b], xj_hbm.at[pl.ds(0, G_CH * LANE), pl.ds(0, IN)],
            osems[b]).wait()


_gather = functools.partial(
    pl.kernel,
    out_type=jax.ShapeDtypeStruct((E_PAD, 128), _f32),
    mesh=plsc.VectorSubcoreMesh(core_axis_name="c", subcore_axis_name="s"),
    scratch_types=[
        pltpu.VMEM((GA, LANE), jnp.int32),
        pltpu.VMEM((G_CH * LANE, IN), _f32),
        pltpu.VMEM((G_CH * LANE, IN), _f32),
        pltpu.SemaphoreType.DMA,
        pltpu.SemaphoreType.DMA,
        pltpu.SemaphoreType.DMA,
    ],
    compiler_params=pltpu.CompilerParams(use_tc_tiling_on_sc=False),
)(_gather_body)


def _msgs_body(ea, xj, w1, b1, w2, b2, rmat, smat, out):
    h = jnp.maximum(
        jnp.dot(ea[...], w1[...], preferred_element_type=_f32) + b1[...], 0.0
    )
    wflat = jnp.dot(h, w2[...], preferred_element_type=_f32) + b2[...]
    xt = jnp.dot(xj[:, 0:IN], rmat[...], preferred_element_type=_f32)
    out[:, 0:OUT] = jnp.dot(xt * wflat, smat[...],
                            preferred_element_type=_f32)


def _msgs(ea, xj, W1, b1, W2, b2, rmat, smat):
    full = lambda shape: pl.BlockSpec(shape, lambda i: (0, 0))
    return pl.pallas_call(
        _msgs_body,
        grid=(NB,),
        in_specs=[
            pl.BlockSpec((BE, D_EDGE), lambda i: (i, 0)),
            pl.BlockSpec((BE, 128), lambda i: (i, 0)),
            full((D_EDGE, HID)),
            full((1, HID)),
            full((HID, IN * OUT)),
            full((1, IN * OUT)),
            full((IN, IN * OUT)),
            full((IN * OUT, OUT)),
        ],
        out_specs=pl.BlockSpec((BE, 128), lambda i: (i, 0)),
        out_shape=jax.ShapeDtypeStruct((E_PAD, 128), _f32),
        compiler_params=pltpu.CompilerParams(
            dimension_semantics=("parallel",)
        ),
    )(ea, xj, W1, b1, W2, b2, rmat, smat)


def _counts_body(dstidx_hbm, cnts_hbm, idx_v, onesb, zbuf, cacc, csem):
    c = lax.axis_index("c")
    s = lax.axis_index("s")
    w = s * NC + c

    def fill(i, carry):
        zbuf[i] = jnp.zeros((OUT,), _f32)
        return carry

    lax.fori_loop(0, STRIPE, fill, 0)

    def fill1(i, carry):
        onesb[i] = jnp.ones((OUT,), _f32)
        return carry

    lax.fori_loop(0, LANE, fill1, 0)
    pltpu.sync_copy(zbuf, cacc.at[pl.ds(s * STRIPE, STRIPE)])
    pltpu.sync_copy(dstidx_hbm.at[w], idx_v)
    plsc.subcore_barrier()

    def chunk(k, carry):
        for r in range(S_CH):
            pltpu.async_copy(
                onesb, cacc.at[idx_v.at[k * S_CH + r]], csem, add=True)
        for r in range(S_CH):
            pltpu.make_async_copy(
                cnts_hbm.at[0, pl.ds(0, LANE), pl.ds(0, OUT)], onesb,
                csem).wait()
        return carry

    lax.fori_loop(0, RPW // S_CH, chunk, 0)
    plsc.subcore_barrier()
    pltpu.sync_copy(cacc.at[pl.ds(s * STRIPE, STRIPE)],
                    cnts_hbm.at[c, pl.ds(s * STRIPE, STRIPE), pl.ds(0, OUT)])


_counts = functools.partial(
    pl.kernel,
    out_type=jax.ShapeDtypeStruct((NC, N_ACC, 128), _f32),
    mesh=plsc.VectorSubcoreMesh(core_axis_name="c", subcore_axis_name="s"),
    scratch_types=[
        pltpu.VMEM((RPW, LANE), jnp.int32),
        pltpu.VMEM((LANE, OUT), _f32),
        pltpu.VMEM((STRIPE, OUT), _f32),
        pltpu.VMEM_SHARED((N_ACC, OUT), _f32),
        pltpu.SemaphoreType.DMA,
    ],
    compiler_params=pltpu.CompilerParams(use_tc_tiling_on_sc=False),
)(_counts_body)


def _scatter_body(msgs_hbm, dstidx_hbm, sums_hbm,
                  idx_v, mbuf0, mbuf1, zbuf, acc, ssem0, ssem1):
    c = lax.axis_index("c")
    s = lax.axis_index("s")
    w = s * NC + c

    def fill(i, carry):
        zbuf[i] = jnp.zeros((OUT,), _f32)
        return carry

    lax.fori_loop(0, STRIPE, fill, 0)
    pltpu.sync_copy(zbuf, acc.at[pl.ds(s * STRIPE, STRIPE)])
    pltpu.sync_copy(dstidx_hbm.at[w], idx_v)
    plsc.subcore_barrier()
    mbufs = (mbuf0, mbuf1)
    ssems = (ssem0, ssem1)

    def outer(k2, carry):
        for b in range(2):
            kk = k2 * 2 + b
            mb = mbufs[b]
            ss = ssems[b]

            @pl.when(kk >= 2)
            def _drain():
                pltpu.make_async_copy(
                    msgs_hbm.at[pl.ds(0, S_CH * LANE), pl.ds(0, OUT)],
                    mb, ss).wait()

            pltpu.sync_copy(
                msgs_hbm.at[pl.ds((w * RPW + kk * S_CH) * LANE,
                                  S_CH * LANE), pl.ds(0, OUT)], mb)
            for r in range(S_CH):
                pltpu.async_copy(
                    mb.at[pl.ds(r * LANE, LANE)],
                    acc.at[idx_v.at[kk * S_CH + r]],
                    ss,
                    add=True,
                )
        return carry

    lax.fori_loop(0, RPW // S_CH // 2, outer, 0)
    for b in range(2):
        pltpu.make_async_copy(
            msgs_hbm.at[pl.ds(0, S_CH * LANE), pl.ds(0, OUT)],
            mbufs[b], ssems[b]).wait()
    plsc.subcore_barrier()
    pltpu.sync_copy(acc.at[pl.ds(s * STRIPE, STRIPE)],
                    sums_hbm.at[c, pl.ds(s * STRIPE, STRIPE), pl.ds(0, OUT)])


_scatter = functools.partial(
    pl.kernel,
    out_type=jax.ShapeDtypeStruct((NC, N_ACC, 128), _f32),
    mesh=plsc.VectorSubcoreMesh(core_axis_name="c", subcore_axis_name="s"),
    scratch_types=[
        pltpu.VMEM((RPW, LANE), jnp.int32),
        pltpu.VMEM((S_CH * LANE, OUT), _f32),
        pltpu.VMEM((S_CH * LANE, OUT), _f32),
        pltpu.VMEM((STRIPE, OUT), _f32),
        pltpu.VMEM_SHARED((N_ACC, OUT), _f32),
        pltpu.SemaphoreType.DMA,
        pltpu.SemaphoreType.DMA,
    ],
    compiler_params=pltpu.CompilerParams(use_tc_tiling_on_sc=False),
)(_scatter_body)


def _final_body(sums_ref, cnts_ref, x_ref, root_ref, bias_ref, gamma_ref,
                beta_ref, out_ref):
    summ = (sums_ref[0] + sums_ref[1])[0:N, 0:OUT]
    cnt = (cnts_ref[0] + cnts_ref[1])[0:N, 0:OUT]
    aggr = summ / jnp.maximum(cnt, 1.0)
    xv = x_ref[...]
    h = aggr + jnp.dot(xv, root_ref[...], preferred_element_type=_f32) \
        + bias_ref[...]
    mu = jnp.mean(h, axis=0, keepdims=True)
    var = jnp.mean((h - mu) ** 2, axis=0, keepdims=True)
    hn = (h - mu) / jnp.sqrt(var + 1e-5) * gamma_ref[...] + beta_ref[...]
    out_ref[...] = xv + jnp.maximum(hn, 0.0)


def _final(sums, cnts, x, root, bias, gamma, beta):
    return pl.pallas_call(
        _final_body,
        out_shape=jax.ShapeDtypeStruct((N, OUT), _f32),
    )(sums, cnts, x, root, bias, gamma, beta)


def kernel(x, edge_index, edge_attr, W1, b1, W2, b2, root, bias, gamma, beta):
    src = edge_index[0].astype(jnp.int32)
    dst = edge_index[1].astype(jnp.int32)
    src_p = jnp.concatenate(
        [src, jnp.zeros((GIDX * LANE - E,), jnp.int32)]).reshape(GIDX, LANE)
    dst_p = jnp.concatenate(
        [dst, jnp.full((PAD,), N, jnp.int32)]).reshape(NW, RPW, LANE)

    cols = jnp.arange(IN * OUT)
    rmat = (cols[None, :] // OUT == jnp.arange(IN)[:, None]).astype(_f32)
    smat = (cols[:, None] % OUT == jnp.arange(OUT)[None, :]).astype(_f32)

    cnts = _counts(dst_p)
    xj = _gather(x, src_p)
    msgs = _msgs(edge_attr, xj, W1, b1.reshape(1, HID), W2,
                 b2.reshape(1, IN * OUT), rmat, smat)
    sums = _scatter(msgs, dst_p)

    return _final(sums, cnts, x, root,
                  bias.reshape(1, OUT), gamma.reshape(1, OUT),
                  beta.reshape(1, OUT))
